# baseline (speedup 1.0000x reference)
import jax
import jax.numpy as jnp
from jax import lax
from jax.experimental import pallas as pl
from jax.experimental.pallas import tpu as pltpu

_P = 8732
_C = 21
_RB = 4


def _smooth_l1(d, posm3):
    a = jnp.abs(d)
    l = jnp.where(a < 1.0, 0.5 * d * d, a - 0.5)
    return jnp.sum(jnp.where(posm3, l, 0.0))


def _fused(conf_ref, ct_ref, dloc_ref, dsz_ref, dori_ref, sloc_ref,
           sori_ref, ssz_ref, ces_ref, nsum_ref):
    g = pl.program_id(0)

    conf = conf_ref[...]
    ct = ct_ref[:, 0, :]

    m = jnp.max(conf, axis=1)
    s = jnp.sum(jnp.exp(conf - m[:, None, :]), axis=1)
    lse = m + jnp.log(s)
    picked = jnp.where(ct > 0, conf[:, 1, :], conf[:, 0, :])
    ce = lse - picked
    lc = jnp.where(ct > 0, 0.0, ce)

    posm = ct > 1
    npos = jnp.sum(posm.astype(jnp.int32), axis=1, keepdims=True)
    k = jnp.minimum(3 * npos, _P - 1)

    bits = lax.bitcast_convert_type(lc, jnp.int32)
    c_gt0 = jnp.sum((bits > 0).astype(jnp.int32), axis=1, keepdims=True)

    def search(_):
        def body(i, u):
            cand = u | lax.shift_left(jnp.int32(1), 30 - i)
            cnt = jnp.sum((bits >= cand).astype(jnp.int32), axis=1,
                          keepdims=True)
            return jnp.where(cnt >= k, cand, u)

        return lax.fori_loop(0, 31, body, jnp.zeros_like(k))

    t = lax.cond(jnp.any(c_gt0 >= k), search,
                 lambda _: jnp.zeros_like(k), 0)

    gt = bits > t
    c_gt = jnp.sum(gt.astype(jnp.int32), axis=1, keepdims=True)
    need = k - c_gt
    eq = (bits == t).astype(jnp.int32)
    ps = eq
    lane = lax.broadcasted_iota(jnp.int32, ps.shape, 1)
    d = 1
    while d < _P:
        ps = ps + jnp.where(lane >= d, pltpu.roll(ps, d, 1), 0)
        d *= 2
    neg = gt | ((eq > 0) & ((ps - eq) < need))
    sel = neg | posm
    ce_rows = jnp.sum(jnp.where(sel, ce, 0.0))

    @pl.when(g == 0)
    def _():
        sloc_ref[...] = jnp.zeros_like(sloc_ref)
        sori_ref[...] = jnp.zeros_like(sori_ref)
        ssz_ref[...] = jnp.zeros_like(ssz_ref)
        ces_ref[...] = jnp.zeros_like(ces_ref)
        nsum_ref[...] = jnp.zeros_like(nsum_ref)

    posm3 = posm[:, None, :]
    sloc_ref[...] += _smooth_l1(dloc_ref[...], posm3)
    sori_ref[...] += _smooth_l1(dori_ref[...], posm3)
    ssz_ref[...] += _smooth_l1(dsz_ref[...], posm3)
    ces_ref[...] += ce_rows
    nsum_ref[...] += jnp.sum(npos)


@jax.jit
def kernel(loc_pred, conf_pred, size_tr_pred, ori_pred, priors, conf_t,
           loc_t, size_tr_t, ori_t):
    del priors
    B = conf_pred.shape[0]
    tr = lambda x: jnp.transpose(x, (0, 2, 1))

    f32 = jnp.float32
    sc = pl.BlockSpec((1, 1), lambda g: (0, 0))
    bk = lambda k: pl.BlockSpec((_RB, k, _P), lambda g: (g, 0, 0))

    sloc, sori, ssz, ces, nsum = pl.pallas_call(
        _fused,
        grid=(B // _RB,),
        in_specs=[bk(_C), bk(1), bk(4), bk(6), bk(2)],
        out_specs=[sc, sc, sc, sc, sc],
        out_shape=[
            jax.ShapeDtypeStruct((1, 1), f32),
            jax.ShapeDtypeStruct((1, 1), f32),
            jax.ShapeDtypeStruct((1, 1), f32),
            jax.ShapeDtypeStruct((1, 1), f32),
            jax.ShapeDtypeStruct((1, 1), jnp.int32),
        ],
    )(tr(conf_pred), conf_t.reshape(B, 1, _P), tr(loc_pred - loc_t),
      tr(size_tr_pred - size_tr_t), tr(ori_pred - ori_t))

    N = nsum[0, 0].astype(f32)
    return (sloc[0, 0], sori[0, 0], ssz[0, 0], ces[0, 0] / N, N)


# speedup vs baseline: 1.0905x; 1.0905x over previous
import jax
import jax.numpy as jnp
from jax import lax
from jax.experimental import pallas as pl
from jax.experimental.pallas import tpu as pltpu

_P = 8732
_C = 21
_RB = 8


def _smooth_l1(pred, tgt, posm3):
    d = pred - tgt
    a = jnp.abs(d)
    l = jnp.where(a < 1.0, 0.5 * d * d, a - 0.5)
    return jnp.sum(jnp.where(posm3, l, 0.0))


def _fused(conf_ref, ct_ref, locp_ref, loct_ref, szp_ref, szt_ref,
           orp_ref, ort_ref, sloc_ref, sori_ref, ssz_ref, ces_ref,
           nsum_ref):
    g = pl.program_id(0)

    conf = conf_ref[...]
    ct = ct_ref[:, 0, :]

    m = jnp.max(conf, axis=1)
    s = jnp.sum(jnp.exp(conf - m[:, None, :]), axis=1)
    lse = m + jnp.log(s)
    picked = jnp.where(ct > 0, conf[:, 1, :], conf[:, 0, :])
    ce = lse - picked
    lc = jnp.where(ct > 0, 0.0, ce)

    posm = ct > 1
    npos = jnp.sum(posm.astype(jnp.int32), axis=1, keepdims=True)
    k = jnp.minimum(3 * npos, _P - 1)

    bits = lax.bitcast_convert_type(lc, jnp.int32)
    c_gt0 = jnp.sum((bits > 0).astype(jnp.int32), axis=1, keepdims=True)

    def search(_):
        def body(i, u):
            cand = u | lax.shift_left(jnp.int32(1), 30 - i)
            cnt = jnp.sum((bits >= cand).astype(jnp.int32), axis=1,
                          keepdims=True)
            return jnp.where(cnt >= k, cand, u)

        return lax.fori_loop(0, 31, body, jnp.zeros_like(k))

    t = lax.cond(jnp.any(c_gt0 >= k), search,
                 lambda _: jnp.zeros_like(k), 0)

    gt = bits > t
    c_gt = jnp.sum(gt.astype(jnp.int32), axis=1, keepdims=True)
    need = k - c_gt
    eq = (bits == t).astype(jnp.int32)
    ps = eq
    lane = lax.broadcasted_iota(jnp.int32, ps.shape, 1)
    d = 1
    while d < _P:
        ps = ps + jnp.where(lane >= d, pltpu.roll(ps, d, 1), 0)
        d *= 2
    neg = gt | ((eq > 0) & ((ps - eq) < need))
    sel = neg | posm
    ce_rows = jnp.sum(jnp.where(sel, ce, 0.0))

    @pl.when(g == 0)
    def _():
        sloc_ref[...] = jnp.zeros_like(sloc_ref)
        sori_ref[...] = jnp.zeros_like(sori_ref)
        ssz_ref[...] = jnp.zeros_like(ssz_ref)
        ces_ref[...] = jnp.zeros_like(ces_ref)
        nsum_ref[...] = jnp.zeros_like(nsum_ref)

    posm3 = posm[:, None, :]
    sloc_ref[...] += _smooth_l1(locp_ref[...], loct_ref[...], posm3)
    sori_ref[...] += _smooth_l1(orp_ref[...], ort_ref[...], posm3)
    ssz_ref[...] += _smooth_l1(szp_ref[...], szt_ref[...], posm3)
    ces_ref[...] += ce_rows
    nsum_ref[...] += jnp.sum(npos)


@jax.jit
def kernel(loc_pred, conf_pred, size_tr_pred, ori_pred, priors, conf_t,
           loc_t, size_tr_t, ori_t):
    del priors
    B = conf_pred.shape[0]
    tr = lambda x: jnp.transpose(x, (0, 2, 1))

    f32 = jnp.float32
    sc = pl.BlockSpec((1, 1), lambda g: (0, 0))
    bk = lambda k: pl.BlockSpec((_RB, k, _P), lambda g: (g, 0, 0))

    sloc, sori, ssz, ces, nsum = pl.pallas_call(
        _fused,
        grid=(B // _RB,),
        in_specs=[bk(_C), bk(1), bk(4), bk(4), bk(6), bk(6), bk(2), bk(2)],
        out_specs=[sc, sc, sc, sc, sc],
        out_shape=[
            jax.ShapeDtypeStruct((1, 1), f32),
            jax.ShapeDtypeStruct((1, 1), f32),
            jax.ShapeDtypeStruct((1, 1), f32),
            jax.ShapeDtypeStruct((1, 1), f32),
            jax.ShapeDtypeStruct((1, 1), jnp.int32),
        ],
    )(tr(conf_pred), conf_t.reshape(B, 1, _P), tr(loc_pred), tr(loc_t),
      tr(size_tr_pred), tr(size_tr_t), tr(ori_pred), tr(ori_t))

    N = nsum[0, 0].astype(f32)
    return (sloc[0, 0], sori[0, 0], ssz[0, 0], ces[0, 0] / N, N)
